# baseline (speedup 1.0000x reference)
import functools

import jax
import jax.numpy as jnp
from jax import lax
from jax.experimental import pallas as pl
from jax.experimental.pallas import tpu as pltpu
from jax.experimental.pallas import tpu_sc as plsc

N = 10000
D = 128
NC = 2
NS = 16
NW = NC * NS
CHUNK = 128
NCH = 80
GB = 16
NG = NCH // GB
TOTCH = NW * NCH
EPAD = TOTCH * CHUNK
C0 = 48
C1 = NCH * 2 - C0
NPAD = 10240
ZROWS = NPAD // NS
OROWS = N // NS

def _deg_body(dst_hbm, ones_hbm, out_hbm, dst_v, ones_v, zs_v, acc_sh):
    c = lax.axis_index("c")
    s = lax.axis_index("s")
    wid = c * NS + s
    pltpu.sync_copy(dst_hbm.at[pl.ds(wid * NCH, NCH)], dst_v)
    pltpu.sync_copy(ones_hbm, ones_v)

    @pl.loop(0, ZROWS // 16)
    def _zero(i):
        zs_v[pl.ds(i * 16, 16)] = jnp.zeros((16,), jnp.float32)

    pltpu.sync_copy(zs_v, acc_sh.at[pl.ds(s * ZROWS, ZROWS)])
    plsc.subcore_barrier()

    @pl.loop(0, NCH)
    def _scatter(i):
        pltpu.sync_copy(ones_v, acc_sh.at[dst_v.at[i]], add=True)

    plsc.subcore_barrier()
    pltpu.sync_copy(acc_sh.at[pl.ds(s * ZROWS, ZROWS)],
                    out_hbm.at[c, pl.ds(s * ZROWS, ZROWS)])


def _edge_body(y_hbm, src_hbm, dst_hbm, zero_hbm, out_hbm,
               src_v, dst_v, rows_v, acc_sh, sem0, sem1):
    c = lax.axis_index("c")
    s = lax.axis_index("s")
    cbase = jnp.where(c == 0, s * C0, NS * C0 + s * C1)
    ncg = jnp.where(c == 0, C0 // GB, C1 // GB)
    @pl.loop(0, ZROWS // CHUNK)
    def _zero(j):
        pltpu.sync_copy(zero_hbm, acc_sh.at[pl.ds(s * ZROWS + j * CHUNK, CHUNK)])

    plsc.subcore_barrier()

    H = CHUNK // 2

    def _issue(i, base, sem):
        pltpu.async_copy(y_hbm.at[src_v.at[i, pl.ds(0, H)]],
                         rows_v.at[pl.ds(base, H)], sem)
        pltpu.async_copy(y_hbm.at[src_v.at[i, pl.ds(H, H)]],
                         rows_v.at[pl.ds(base + H, H)], sem)

    def _drain(i, base, sem):
        pltpu.make_async_copy(y_hbm.at[src_v.at[i, pl.ds(0, H)]],
                              rows_v.at[pl.ds(base, H)], sem).wait()
        pltpu.make_async_copy(y_hbm.at[src_v.at[i, pl.ds(H, H)]],
                              rows_v.at[pl.ds(base + H, H)], sem).wait()

    @pl.loop(0, ncg)
    def _grp(g):
        off = pl.multiple_of(cbase + g * GB, 8)
        pltpu.sync_copy(src_hbm.at[pl.ds(off, GB)], src_v)
        pltpu.sync_copy(dst_hbm.at[pl.ds(off, GB)], dst_v)
        _issue(0, 0, sem0)
        _issue(1, CHUNK, sem1)

        @pl.loop(0, GB)
        def _body(i):
            odd = lax.rem(i, 2)

            @pl.when(odd == 0)
            def _():
                _drain(i, 0, sem0)
                pltpu.sync_copy(rows_v.at[pl.ds(0, CHUNK)],
                                acc_sh.at[dst_v.at[i]], add=True)

                @pl.when(i + 2 < GB)
                def _():
                    _issue(i + 2, 0, sem0)

            @pl.when(odd == 1)
            def _():
                _drain(i, CHUNK, sem1)
                pltpu.sync_copy(rows_v.at[pl.ds(CHUNK, CHUNK)],
                                acc_sh.at[dst_v.at[i]], add=True)

                @pl.when(i + 2 < GB)
                def _():
                    _issue(i + 2, CHUNK, sem1)

    plsc.subcore_barrier()
    pltpu.sync_copy(acc_sh.at[pl.ds(s * ZROWS, ZROWS)],
                    out_hbm.at[c, pl.ds(s * ZROWS, ZROWS)])


@functools.cache
def _sc_kernels():
    mesh = plsc.VectorSubcoreMesh(core_axis_name="c", subcore_axis_name="s",
                                  num_cores=NC, num_subcores=NS)
    deg = pl.kernel(
        _deg_body,
        out_type=jax.ShapeDtypeStruct((NC, NPAD), jnp.float32),
        mesh=mesh,
        scratch_types=[
            pltpu.VMEM((NCH, CHUNK), jnp.int32),
            pltpu.VMEM((CHUNK,), jnp.float32),
            pltpu.VMEM((ZROWS,), jnp.float32),
            pltpu.VMEM_SHARED((NPAD,), jnp.float32),
        ],
    )
    edge = pl.kernel(
        _edge_body,
        out_type=jax.ShapeDtypeStruct((NC, NPAD, D), jnp.float32),
        mesh=mesh,
        scratch_types=[
            pltpu.VMEM((GB, CHUNK), jnp.int32),
            pltpu.VMEM((GB, CHUNK), jnp.int32),
            pltpu.VMEM((2 * CHUNK, D), jnp.float32),
            pltpu.VMEM_SHARED((NPAD, D), jnp.float32),
            pltpu.SemaphoreType.DMA,
            pltpu.SemaphoreType.DMA,
        ],
    )
    return deg, edge


BN = 1000
_GRID = N // BN


def _tc_b_body(degp_ref, x_ref, w_ref, dis_ref, y_ref):
    dp = degp_ref[...]
    deg = 1.0 + dp[0] + dp[1]
    dis = lax.rsqrt(deg)
    dis_ref[...] = dis
    y_ref[...] = jnp.dot(x_ref[...], w_ref[...],
                         preferred_element_type=jnp.float32) * dis


def _tc_b(degp, x, w1):
    return pl.pallas_call(
        _tc_b_body,
        grid=(_GRID,),
        in_specs=[
            pl.BlockSpec((2, BN, 1), lambda i: (0, i, 0)),
            pl.BlockSpec((BN, D), lambda i: (i, 0)),
            pl.BlockSpec((D, D), lambda i: (0, 0)),
        ],
        out_specs=[
            pl.BlockSpec((BN, 1), lambda i: (i, 0)),
            pl.BlockSpec((BN, D), lambda i: (i, 0)),
        ],
        out_shape=[
            jax.ShapeDtypeStruct((N, 1), jnp.float32),
            jax.ShapeDtypeStruct((N, D), jnp.float32),
        ],
    )(degp, x, w1)


def _tc_d_body(p_ref, y_ref, dis_ref, b_ref, w2_ref, y2_ref):
    p = p_ref[...]
    dis = dis_ref[...]
    h = jax.nn.relu(dis * (p[0] + p[1] + y_ref[...]) + b_ref[...])
    y2_ref[...] = jnp.dot(h, w2_ref[...],
                          preferred_element_type=jnp.float32) * dis


def _tc_d(p1, y1, dis, b1, w2):
    return pl.pallas_call(
        _tc_d_body,
        grid=(_GRID,),
        in_specs=[
            pl.BlockSpec((2, BN, D), lambda i: (0, i, 0)),
            pl.BlockSpec((BN, D), lambda i: (i, 0)),
            pl.BlockSpec((BN, 1), lambda i: (i, 0)),
            pl.BlockSpec((1, D), lambda i: (0, 0)),
            pl.BlockSpec((D, D), lambda i: (0, 0)),
        ],
        out_specs=pl.BlockSpec((BN, D), lambda i: (i, 0)),
        out_shape=jax.ShapeDtypeStruct((N, D), jnp.float32),
    )(p1, y1, dis, b1, w2)


def _tc_f_body(p_ref, y_ref, dis_ref, b_ref,
               a1_ref, a1b_ref, a2_ref, a2b_ref, a3_ref, a3b_ref,
               v1_ref, v1b_ref, v2_ref, v2b_ref, v3_ref, v3b_ref, q_ref):
    p = p_ref[...]
    dis = dis_ref[...]
    h = jax.nn.relu(dis * (p[0] + p[1] + y_ref[...]) + b_ref[...])
    f32 = jnp.float32
    ta = jax.nn.relu(jnp.dot(h, a1_ref[...], preferred_element_type=f32)
                     + a1b_ref[...])
    ta = jax.nn.relu(jnp.dot(ta, a2_ref[...], preferred_element_type=f32)
                     + a2b_ref[...])
    adv = jnp.dot(ta, a3_ref[...], preferred_element_type=f32) + a3b_ref[...]
    tv = jax.nn.relu(jnp.dot(h, v1_ref[...], preferred_element_type=f32)
                     + v1b_ref[...])
    tv = jax.nn.relu(jnp.dot(tv, v2_ref[...], preferred_element_type=f32)
                     + v2b_ref[...])
    val = jnp.dot(tv, v3_ref[...], preferred_element_type=f32) + v3b_ref[...]
    q_ref[...] = val + adv - jnp.mean(adv, axis=-1, keepdims=True)


def _tc_f(p2, y2, dis, b2, A1, a1b, A2, a2b, A3, a3b, V1, v1b, V2, v2b, V3, v3b):
    full = lambda shape: pl.BlockSpec(shape, lambda i: tuple(0 for _ in shape))
    return pl.pallas_call(
        _tc_f_body,
        grid=(_GRID,),
        in_specs=[
            pl.BlockSpec((2, BN, D), lambda i: (0, i, 0)),
            pl.BlockSpec((BN, D), lambda i: (i, 0)),
            pl.BlockSpec((BN, 1), lambda i: (i, 0)),
            full((1, D)),
            full((D, D)), full((1, D)),
            full((D, D)), full((1, D)),
            full((D, 5)), full((1, 5)),
            full((D, D)), full((1, D)),
            full((D, D)), full((1, D)),
            full((D, 1)), full((1, 1)),
        ],
        out_specs=pl.BlockSpec((BN, 5), lambda i: (i, 0)),
        out_shape=jax.ShapeDtypeStruct((N, 5), jnp.float32),
    )(p2, y2, dis, b2, A1, a1b, A2, a2b, A3, a3b, V1, v1b, V2, v2b, V3, v3b)


def kernel(x, edge_index, W_gcn1, b_gcn1, W_gcn2, b_gcn2,
           A1, a1b, A2, a2b, A3, a3b, V1, v1b, V2, v2b, V3, v3b):
    f32 = jnp.float32
    src = edge_index[0]
    dst = edge_index[1]
    e = src.shape[0]
    pad = EPAD - e
    src_p = jnp.concatenate([src, jnp.zeros((pad,), jnp.int32)]
                            ).reshape(TOTCH, CHUNK)
    dst_p = jnp.concatenate([dst, jnp.full((pad,), N, jnp.int32)]
                            ).reshape(TOTCH, CHUNK)
    ones_e = jnp.ones((CHUNK,), f32)
    zeros_rows = jnp.zeros((CHUNK, D), f32)

    deg_kernel, edge_kernel = _sc_kernels()
    degp = deg_kernel(dst_p, ones_e)[:, :N, None]
    dis, y1 = _tc_b(degp, x, W_gcn1)
    p1 = edge_kernel(y1, src_p, dst_p, zeros_rows)[:, :N]
    y2 = _tc_d(p1, y1, dis, b_gcn1[None, :], W_gcn2)
    p2 = edge_kernel(y2, src_p, dst_p, zeros_rows)[:, :N]
    q = _tc_f(p2, y2, dis, b_gcn2[None, :],
              A1, a1b[None, :], A2, a2b[None, :], A3, a3b[None, :],
              V1, v1b[None, :], V2, v2b[None, :], V3, v3b[None, :])
    return q


# speedup vs baseline: 1.4358x; 1.4358x over previous
import functools

import jax
import jax.numpy as jnp
from jax import lax
from jax.experimental import pallas as pl
from jax.experimental.pallas import tpu as pltpu
from jax.experimental.pallas import tpu_sc as plsc

N = 10000
D = 128
NC = 2
NS = 16
NW = NC * NS
CHUNK = 128
NCH = 80
GB = 16
NG = NCH // GB
TOTCH = NW * NCH
EPAD = TOTCH * CHUNK
C0 = 120
C1 = NCH * 2 - C0
NPAD = 10240
ZROWS = NPAD // NS
OROWS = N // NS

def _deg_body(dst_hbm, ones_hbm, out_hbm, dst_v, ones_v, zs_v, acc_sh):
    c = lax.axis_index("c")
    s = lax.axis_index("s")
    wid = c * NS + s
    pltpu.sync_copy(dst_hbm.at[pl.ds(wid * NCH, NCH)], dst_v)
    pltpu.sync_copy(ones_hbm, ones_v)

    @pl.loop(0, ZROWS // 16)
    def _zero(i):
        zs_v[pl.ds(i * 16, 16)] = jnp.zeros((16,), jnp.float32)

    pltpu.sync_copy(zs_v, acc_sh.at[pl.ds(s * ZROWS, ZROWS)])
    plsc.subcore_barrier()

    @pl.loop(0, NCH)
    def _scatter(i):
        pltpu.sync_copy(ones_v, acc_sh.at[dst_v.at[i]], add=True)

    plsc.subcore_barrier()
    pltpu.sync_copy(acc_sh.at[pl.ds(s * ZROWS, ZROWS)],
                    out_hbm.at[c, pl.ds(s * ZROWS, ZROWS)])


def _edge_body(y_hbm, src_hbm, dst_hbm, zero_hbm, out_hbm,
               src_v, dst_v, rows_v, acc_sh, sem0, sem1):
    c = lax.axis_index("c")
    s = lax.axis_index("s")
    cbase = jnp.where(c == 0, s * C0, NS * C0 + s * C1)
    ncg = jnp.where(c == 0, C0 // GB, C1 // GB)
    @pl.loop(0, ZROWS // CHUNK)
    def _zero(j):
        pltpu.sync_copy(zero_hbm, acc_sh.at[pl.ds(s * ZROWS + j * CHUNK, CHUNK)])

    plsc.subcore_barrier()

    H = CHUNK // 2

    def _issue(i, base, sem):
        pltpu.async_copy(y_hbm.at[src_v.at[i, pl.ds(0, H)]],
                         rows_v.at[pl.ds(base, H)], sem)
        pltpu.async_copy(y_hbm.at[src_v.at[i, pl.ds(H, H)]],
                         rows_v.at[pl.ds(base + H, H)], sem)

    def _drain(i, base, sem):
        pltpu.make_async_copy(y_hbm.at[src_v.at[i, pl.ds(0, H)]],
                              rows_v.at[pl.ds(base, H)], sem).wait()
        pltpu.make_async_copy(y_hbm.at[src_v.at[i, pl.ds(H, H)]],
                              rows_v.at[pl.ds(base + H, H)], sem).wait()

    @pl.loop(0, ncg)
    def _grp(g):
        off = pl.multiple_of(cbase + g * GB, 8)
        pltpu.sync_copy(src_hbm.at[pl.ds(off, GB)], src_v)
        pltpu.sync_copy(dst_hbm.at[pl.ds(off, GB)], dst_v)
        _issue(0, 0, sem0)
        _issue(1, CHUNK, sem1)

        @pl.loop(0, GB)
        def _body(i):
            odd = lax.rem(i, 2)

            @pl.when(odd == 0)
            def _():
                _drain(i, 0, sem0)
                pltpu.sync_copy(rows_v.at[pl.ds(0, CHUNK)],
                                acc_sh.at[dst_v.at[i]], add=True)

                @pl.when(i + 2 < GB)
                def _():
                    _issue(i + 2, 0, sem0)

            @pl.when(odd == 1)
            def _():
                _drain(i, CHUNK, sem1)
                pltpu.sync_copy(rows_v.at[pl.ds(CHUNK, CHUNK)],
                                acc_sh.at[dst_v.at[i]], add=True)

                @pl.when(i + 2 < GB)
                def _():
                    _issue(i + 2, CHUNK, sem1)

    plsc.subcore_barrier()
    pltpu.sync_copy(acc_sh.at[pl.ds(s * ZROWS, ZROWS)],
                    out_hbm.at[c, pl.ds(s * ZROWS, ZROWS)])


@functools.cache
def _sc_kernels():
    mesh = plsc.VectorSubcoreMesh(core_axis_name="c", subcore_axis_name="s",
                                  num_cores=NC, num_subcores=NS)
    deg = pl.kernel(
        _deg_body,
        out_type=jax.ShapeDtypeStruct((NC, NPAD), jnp.float32),
        mesh=mesh,
        scratch_types=[
            pltpu.VMEM((NCH, CHUNK), jnp.int32),
            pltpu.VMEM((CHUNK,), jnp.float32),
            pltpu.VMEM((ZROWS,), jnp.float32),
            pltpu.VMEM_SHARED((NPAD,), jnp.float32),
        ],
    )
    edge = pl.kernel(
        _edge_body,
        out_type=jax.ShapeDtypeStruct((NC, NPAD, D), jnp.float32),
        mesh=mesh,
        scratch_types=[
            pltpu.VMEM((GB, CHUNK), jnp.int32),
            pltpu.VMEM((GB, CHUNK), jnp.int32),
            pltpu.VMEM((2 * CHUNK, D), jnp.float32),
            pltpu.VMEM_SHARED((NPAD, D), jnp.float32),
            pltpu.SemaphoreType.DMA,
            pltpu.SemaphoreType.DMA,
        ],
    )
    return deg, edge


BN = 1000
_GRID = N // BN


def _tc_b_body(degp_ref, x_ref, w_ref, dis_ref, y_ref):
    dp = degp_ref[...]
    deg = 1.0 + dp[0] + dp[1]
    dis = lax.rsqrt(deg)
    dis_ref[...] = dis
    y_ref[...] = jnp.dot(x_ref[...], w_ref[...],
                         preferred_element_type=jnp.float32) * dis


def _tc_b(degp, x, w1):
    return pl.pallas_call(
        _tc_b_body,
        grid=(_GRID,),
        in_specs=[
            pl.BlockSpec((2, BN, 1), lambda i: (0, i, 0)),
            pl.BlockSpec((BN, D), lambda i: (i, 0)),
            pl.BlockSpec((D, D), lambda i: (0, 0)),
        ],
        out_specs=[
            pl.BlockSpec((BN, 1), lambda i: (i, 0)),
            pl.BlockSpec((BN, D), lambda i: (i, 0)),
        ],
        out_shape=[
            jax.ShapeDtypeStruct((N, 1), jnp.float32),
            jax.ShapeDtypeStruct((N, D), jnp.float32),
        ],
    )(degp, x, w1)


def _tc_d_body(p_ref, y_ref, dis_ref, b_ref, w2_ref, y2_ref):
    p = p_ref[...]
    dis = dis_ref[...]
    h = jax.nn.relu(dis * (p[0] + p[1] + y_ref[...]) + b_ref[...])
    y2_ref[...] = jnp.dot(h, w2_ref[...],
                          preferred_element_type=jnp.float32) * dis


def _tc_d(p1, y1, dis, b1, w2):
    return pl.pallas_call(
        _tc_d_body,
        grid=(_GRID,),
        in_specs=[
            pl.BlockSpec((2, BN, D), lambda i: (0, i, 0)),
            pl.BlockSpec((BN, D), lambda i: (i, 0)),
            pl.BlockSpec((BN, 1), lambda i: (i, 0)),
            pl.BlockSpec((1, D), lambda i: (0, 0)),
            pl.BlockSpec((D, D), lambda i: (0, 0)),
        ],
        out_specs=pl.BlockSpec((BN, D), lambda i: (i, 0)),
        out_shape=jax.ShapeDtypeStruct((N, D), jnp.float32),
    )(p1, y1, dis, b1, w2)


def _tc_f_body(p_ref, y_ref, dis_ref, b_ref,
               a1_ref, a1b_ref, a2_ref, a2b_ref, a3_ref, a3b_ref,
               v1_ref, v1b_ref, v2_ref, v2b_ref, v3_ref, v3b_ref, q_ref):
    p = p_ref[...]
    dis = dis_ref[...]
    h = jax.nn.relu(dis * (p[0] + p[1] + y_ref[...]) + b_ref[...])
    f32 = jnp.float32
    ta = jax.nn.relu(jnp.dot(h, a1_ref[...], preferred_element_type=f32)
                     + a1b_ref[...])
    ta = jax.nn.relu(jnp.dot(ta, a2_ref[...], preferred_element_type=f32)
                     + a2b_ref[...])
    adv = jnp.dot(ta, a3_ref[...], preferred_element_type=f32) + a3b_ref[...]
    tv = jax.nn.relu(jnp.dot(h, v1_ref[...], preferred_element_type=f32)
                     + v1b_ref[...])
    tv = jax.nn.relu(jnp.dot(tv, v2_ref[...], preferred_element_type=f32)
                     + v2b_ref[...])
    val = jnp.dot(tv, v3_ref[...], preferred_element_type=f32) + v3b_ref[...]
    q_ref[...] = val + adv - jnp.mean(adv, axis=-1, keepdims=True)


def _tc_f(p2, y2, dis, b2, A1, a1b, A2, a2b, A3, a3b, V1, v1b, V2, v2b, V3, v3b):
    full = lambda shape: pl.BlockSpec(shape, lambda i: tuple(0 for _ in shape))
    return pl.pallas_call(
        _tc_f_body,
        grid=(_GRID,),
        in_specs=[
            pl.BlockSpec((2, BN, D), lambda i: (0, i, 0)),
            pl.BlockSpec((BN, D), lambda i: (i, 0)),
            pl.BlockSpec((BN, 1), lambda i: (i, 0)),
            full((1, D)),
            full((D, D)), full((1, D)),
            full((D, D)), full((1, D)),
            full((D, 5)), full((1, 5)),
            full((D, D)), full((1, D)),
            full((D, D)), full((1, D)),
            full((D, 1)), full((1, 1)),
        ],
        out_specs=pl.BlockSpec((BN, 5), lambda i: (i, 0)),
        out_shape=jax.ShapeDtypeStruct((N, 5), jnp.float32),
    )(p2, y2, dis, b2, A1, a1b, A2, a2b, A3, a3b, V1, v1b, V2, v2b, V3, v3b)


def kernel(x, edge_index, W_gcn1, b_gcn1, W_gcn2, b_gcn2,
           A1, a1b, A2, a2b, A3, a3b, V1, v1b, V2, v2b, V3, v3b):
    f32 = jnp.float32
    src = edge_index[0]
    dst = edge_index[1]
    e = src.shape[0]
    pad = EPAD - e
    src_p = jnp.concatenate([src, jnp.zeros((pad,), jnp.int32)]
                            ).reshape(TOTCH, CHUNK)
    dst_p = jnp.concatenate([dst, jnp.full((pad,), N, jnp.int32)]
                            ).reshape(TOTCH, CHUNK)
    ones_e = jnp.ones((CHUNK,), f32)
    zeros_rows = jnp.zeros((CHUNK, D), f32)

    deg_kernel, edge_kernel = _sc_kernels()
    degp = deg_kernel(dst_p, ones_e)[:, :N, None]
    dis, y1 = _tc_b(degp, x, W_gcn1)
    p1 = edge_kernel(y1, src_p, dst_p, zeros_rows)[:, :N]
    y2 = _tc_d(p1, y1, dis, b_gcn1[None, :], W_gcn2)
    p2 = edge_kernel(y2, src_p, dst_p, zeros_rows)[:, :N]
    q = _tc_f(p2, y2, dis, b_gcn2[None, :],
              A1, a1b[None, :], A2, a2b[None, :], A3, a3b[None, :],
              V1, v1b[None, :], V2, v2b[None, :], V3, v3b[None, :])
    return q
